# baseline (speedup 1.0000x reference)
import jax
import jax.numpy as jnp
from jax import lax
from jax.experimental import pallas as pl
from jax.experimental.pallas import tpu as pltpu
from jax.experimental.pallas import tpu_sc as plsc

NUM_ENTITIES = 1000000
NUM_RELATIONS = 1000
EMBED_DIM = 64
BATCH = 16384

NC = 2
NS = 16
NW = NC * NS
B_PER_W = BATCH // NW
CHUNK = 128
NCHUNK = B_PER_W // CHUNK
L = 16
D2 = 2 * EMBED_DIM

SLAB = 160
NSLAB = NUM_ENTITIES // SLAB
KMAX = (NSLAB + NW - 1) // NW
NPAIR = KMAX // 2


def _pair_body(er, ei, rrt, rit, ent2, rel2,
               bra, bia, bca, brb, bib, bcb, sra, srb, swa, swb):
    wid = lax.axis_index("s") * NC + lax.axis_index("c")

    def rd(buf_r, buf_i, s, sem, n=SLAB):
        if n == SLAB:
            dr, di = buf_r, buf_i
        else:
            dr, di = buf_r.at[pl.ds(0, n)], buf_i.at[pl.ds(0, n)]
        return (pltpu.make_async_copy(er.at[pl.ds(s * SLAB, n)], dr, sem),
                pltpu.make_async_copy(ei.at[pl.ds(s * SLAB, n)], di, sem))

    def combine(buf_r, buf_i, buf_c, n=SLAB):
        def row_body(r, carry):
            for j in range(EMBED_DIM // L):
                sl = pl.ds(j * L, L)
                buf_c[r, sl] = buf_r[r, sl]
                buf_c[r, pl.ds(EMBED_DIM + j * L, L)] = buf_i[r, sl]
            return carry
        lax.fori_loop(0, n, row_body, 0)

    @pl.when(wid < 6)
    def _():
        rbase = wid * SLAB
        pltpu.sync_copy(rrt.at[pl.ds(rbase, SLAB)], bra)
        pltpu.sync_copy(rit.at[pl.ds(rbase, SLAB)], bia)
        combine(bra, bia, bca)
        pltpu.sync_copy(bca, rel2.at[pl.ds(rbase, SLAB)])

    @pl.when(wid == 6)
    def _():
        pltpu.sync_copy(rrt.at[pl.ds(960, 40)], bra.at[pl.ds(0, 40)])
        pltpu.sync_copy(rit.at[pl.ds(960, 40)], bia.at[pl.ds(0, 40)])
        combine(bra, bia, bca, 40)
        pltpu.sync_copy(bca.at[pl.ds(0, 40)], rel2.at[pl.ds(960, 40)])

    def pair_iter(p, carry):
        s0 = wid + NW * 2 * p
        s1 = s0 + NW
        for c in rd(bra, bia, s0, sra):
            c.start()

        @pl.when(s1 < NSLAB)
        def _():
            for c in rd(brb, bib, s1, srb):
                c.start()

        for c in rd(bra, bia, s0, sra):
            c.wait()
        combine(bra, bia, bca)
        w = pltpu.make_async_copy(bca, ent2.at[pl.ds(s0 * SLAB, SLAB)], swa)
        w.start()
        w.wait()

        @pl.when(s1 < NSLAB)
        def _():
            for c in rd(brb, bib, s1, srb):
                c.wait()
            combine(brb, bib, bcb)
            w2 = pltpu.make_async_copy(bcb, ent2.at[pl.ds(s1 * SLAB, SLAB)], swb)
            w2.start()
            w2.wait()

        return carry

    lax.fori_loop(0, NPAIR, pair_iter, 0)


def _gather_body(head_r, rel_r, tail_r, ent2, rel2, out_hbm,
                 idx_h, idx_r, idx_t, g_h, g_t, g_r, out_v, sem):
    wid = lax.axis_index("s") * NC + lax.axis_index("c")
    iota = lax.iota(jnp.int32, L)

    def chunk_body(ci, carry):
        base = wid * B_PER_W + ci * CHUNK
        pltpu.sync_copy(head_r.at[pl.ds(base, CHUNK)], idx_h)
        pltpu.sync_copy(rel_r.at[pl.ds(base, CHUNK)], idx_r)
        pltpu.sync_copy(tail_r.at[pl.ds(base, CHUNK)], idx_t)
        cps = [
            pltpu.async_copy(ent2.at[idx_h], g_h, sem),
            pltpu.async_copy(ent2.at[idx_t], g_t, sem),
            pltpu.async_copy(rel2.at[idx_r], g_r, sem),
        ]
        for cp in cps:
            cp.wait()

        def group_body(g, carry2):
            tot = jnp.zeros((L,), jnp.float32)
            for row in range(L):
                c = g * L + row
                acc = jnp.zeros((L,), jnp.float32)
                for j in range(EMBED_DIM // L):
                    slr = pl.ds(j * L, L)
                    sli = pl.ds(EMBED_DIM + j * L, L)
                    hr = g_h[c, slr]
                    hi = g_h[c, sli]
                    tr = g_t[c, slr]
                    ti = g_t[c, sli]
                    rr = g_r[c, slr]
                    ri = g_r[c, sli]
                    acc = acc + rr * (hr * tr + hi * ti) + ri * (hr * ti - hi * tr)
                s = lax.reduce_sum_p.bind(acc, axes=(0,))
                tot = jnp.where(iota == row, s, tot)
            out_v[pl.ds(g * L, L)] = tot
            return carry2

        lax.fori_loop(0, CHUNK // L, group_body, 0)
        pltpu.sync_copy(out_v, out_hbm.at[pl.ds(base, CHUNK)])
        return carry

    lax.fori_loop(0, NCHUNK, chunk_body, 0)


def kernel(head, relation, tail, ent_real, ent_imag, rel_real, rel_imag):
    mesh = plsc.VectorSubcoreMesh(core_axis_name="c", subcore_axis_name="s")
    cp_ = pltpu.CompilerParams(
        needs_layout_passes=False, use_tc_tiling_on_sc=True)
    pair = pl.kernel(
        _pair_body,
        mesh=mesh,
        compiler_params=cp_,
        out_type=(
            jax.ShapeDtypeStruct((NUM_ENTITIES, D2), jnp.float32),
            jax.ShapeDtypeStruct((NUM_RELATIONS, D2), jnp.float32),
        ),
        scratch_types=[
            pltpu.VMEM((SLAB, EMBED_DIM), jnp.float32),
            pltpu.VMEM((SLAB, EMBED_DIM), jnp.float32),
            pltpu.VMEM((SLAB, D2), jnp.float32),
            pltpu.VMEM((SLAB, EMBED_DIM), jnp.float32),
            pltpu.VMEM((SLAB, EMBED_DIM), jnp.float32),
            pltpu.VMEM((SLAB, D2), jnp.float32),
            pltpu.SemaphoreType.DMA,
            pltpu.SemaphoreType.DMA,
            pltpu.SemaphoreType.DMA,
            pltpu.SemaphoreType.DMA,
        ],
    )
    ent2, rel2 = pair(ent_real, ent_imag, rel_real, rel_imag)
    f = pl.kernel(
        _gather_body,
        mesh=mesh,
        compiler_params=cp_,
        out_type=jax.ShapeDtypeStruct((BATCH,), jnp.float32),
        scratch_types=[
            pltpu.VMEM((CHUNK,), jnp.int32),
            pltpu.VMEM((CHUNK,), jnp.int32),
            pltpu.VMEM((CHUNK,), jnp.int32),
            pltpu.VMEM((CHUNK, D2), jnp.float32),
            pltpu.VMEM((CHUNK, D2), jnp.float32),
            pltpu.VMEM((CHUNK, D2), jnp.float32),
            pltpu.VMEM((CHUNK,), jnp.float32),
            pltpu.SemaphoreType.DMA,
        ],
    )
    return f(head, relation, tail, ent2, rel2)


def probe():
    args = [
        jax.ShapeDtypeStruct((BATCH,), jnp.int32),
        jax.ShapeDtypeStruct((BATCH,), jnp.int32),
        jax.ShapeDtypeStruct((BATCH,), jnp.int32),
        jax.ShapeDtypeStruct((NUM_ENTITIES, EMBED_DIM), jnp.float32),
        jax.ShapeDtypeStruct((NUM_ENTITIES, EMBED_DIM), jnp.float32),
        jax.ShapeDtypeStruct((NUM_RELATIONS, EMBED_DIM), jnp.float32),
        jax.ShapeDtypeStruct((NUM_RELATIONS, EMBED_DIM), jnp.float32),
    ]
    jax.jit(kernel).lower(*args).compile()


# speedup vs baseline: 1.8948x; 1.8948x over previous
import jax
import jax.numpy as jnp
from jax import lax
from jax.experimental import pallas as pl
from jax.experimental.pallas import tpu as pltpu
from jax.experimental.pallas import tpu_sc as plsc

NUM_ENTITIES = 1000000
NUM_RELATIONS = 1000
EMBED_DIM = 64
BATCH = 16384

NC = 2
NS = 16
NW = NC * NS
B_PER_W = BATCH // NW
CHUNK = 128
NCHUNK = B_PER_W // CHUNK
L = 16
D2 = 2 * EMBED_DIM
RSLAB = 160


def _pair_rel_body(rrt, rit, rel2, br, bi, bc):
    wid = lax.axis_index("s") * NC + lax.axis_index("c")

    def combine(n):
        def row_body(r, carry):
            for j in range(EMBED_DIM // L):
                sl = pl.ds(j * L, L)
                bc[r, sl] = br[r, sl]
                bc[r, pl.ds(EMBED_DIM + j * L, L)] = bi[r, sl]
            return carry
        lax.fori_loop(0, n, row_body, 0)

    @pl.when(wid < 6)
    def _():
        rbase = wid * RSLAB
        pltpu.sync_copy(rrt.at[pl.ds(rbase, RSLAB)], br)
        pltpu.sync_copy(rit.at[pl.ds(rbase, RSLAB)], bi)
        combine(RSLAB)
        pltpu.sync_copy(bc, rel2.at[pl.ds(rbase, RSLAB)])

    @pl.when(wid == 6)
    def _():
        pltpu.sync_copy(rrt.at[pl.ds(960, 40)], br.at[pl.ds(0, 40)])
        pltpu.sync_copy(rit.at[pl.ds(960, 40)], bi.at[pl.ds(0, 40)])
        combine(40)
        pltpu.sync_copy(bc.at[pl.ds(0, 40)], rel2.at[pl.ds(960, 40)])


def _gather_body(head_r, rel_r, tail_r, er, ei, rel2, out_hbm,
                 idx_h, idx_r, idx_t, g_hr, g_hi, g_tr, g_ti, g_r,
                 out_v, sem, semr):
    wid = lax.axis_index("s") * NC + lax.axis_index("c")
    iota = lax.iota(jnp.int32, L)

    def chunk_body(ci, carry):
        base = wid * B_PER_W + ci * CHUNK
        pltpu.sync_copy(head_r.at[pl.ds(base, CHUNK)], idx_h)
        pltpu.sync_copy(rel_r.at[pl.ds(base, CHUNK)], idx_r)
        pltpu.sync_copy(tail_r.at[pl.ds(base, CHUNK)], idx_t)
        rel_cp = pltpu.async_copy(rel2.at[idx_r], g_r, semr)

        def fetch_body(g, carry2):
            vh = idx_h[pl.ds(g * L, L)]
            vt = idx_t[pl.ds(g * L, L)]
            cps = []
            for row in range(L):
                r = g * L + row
                ih = vh[row]
                it = vt[row]
                cps.append(pltpu.make_async_copy(er.at[ih], g_hr.at[r], sem))
                cps.append(pltpu.make_async_copy(ei.at[ih], g_hi.at[r], sem))
                cps.append(pltpu.make_async_copy(er.at[it], g_tr.at[r], sem))
                cps.append(pltpu.make_async_copy(ei.at[it], g_ti.at[r], sem))
            for cp in cps:
                cp.start()
            for cp in cps:
                cp.wait()
            return carry2

        lax.fori_loop(0, CHUNK // L, fetch_body, 0)
        rel_cp.wait()

        def group_body(g, carry2):
            tot = jnp.zeros((L,), jnp.float32)
            for row in range(L):
                c = g * L + row
                acc = jnp.zeros((L,), jnp.float32)
                for j in range(EMBED_DIM // L):
                    sl = pl.ds(j * L, L)
                    hr = g_hr[c, sl]
                    hi = g_hi[c, sl]
                    tr = g_tr[c, sl]
                    ti = g_ti[c, sl]
                    rr = g_r[c, sl]
                    ri = g_r[c, pl.ds(EMBED_DIM + j * L, L)]
                    acc = acc + rr * (hr * tr + hi * ti) + ri * (hr * ti - hi * tr)
                s = lax.reduce_sum_p.bind(acc, axes=(0,))
                tot = jnp.where(iota == row, s, tot)
            out_v[pl.ds(g * L, L)] = tot
            return carry2

        lax.fori_loop(0, CHUNK // L, group_body, 0)
        pltpu.sync_copy(out_v, out_hbm.at[pl.ds(base, CHUNK)])
        return carry

    lax.fori_loop(0, NCHUNK, chunk_body, 0)


def kernel(head, relation, tail, ent_real, ent_imag, rel_real, rel_imag):
    mesh = plsc.VectorSubcoreMesh(core_axis_name="c", subcore_axis_name="s")
    cp_ = pltpu.CompilerParams(
        needs_layout_passes=False, use_tc_tiling_on_sc=True)
    pair = pl.kernel(
        _pair_rel_body,
        mesh=mesh,
        compiler_params=cp_,
        out_type=jax.ShapeDtypeStruct((NUM_RELATIONS, D2), jnp.float32),
        scratch_types=[
            pltpu.VMEM((RSLAB, EMBED_DIM), jnp.float32),
            pltpu.VMEM((RSLAB, EMBED_DIM), jnp.float32),
            pltpu.VMEM((RSLAB, D2), jnp.float32),
        ],
    )
    rel2 = pair(rel_real, rel_imag)
    f = pl.kernel(
        _gather_body,
        mesh=mesh,
        compiler_params=cp_,
        out_type=jax.ShapeDtypeStruct((BATCH,), jnp.float32),
        scratch_types=[
            pltpu.VMEM((CHUNK,), jnp.int32),
            pltpu.VMEM((CHUNK,), jnp.int32),
            pltpu.VMEM((CHUNK,), jnp.int32),
            pltpu.VMEM((CHUNK, EMBED_DIM), jnp.float32),
            pltpu.VMEM((CHUNK, EMBED_DIM), jnp.float32),
            pltpu.VMEM((CHUNK, EMBED_DIM), jnp.float32),
            pltpu.VMEM((CHUNK, EMBED_DIM), jnp.float32),
            pltpu.VMEM((CHUNK, D2), jnp.float32),
            pltpu.VMEM((CHUNK,), jnp.float32),
            pltpu.SemaphoreType.DMA,
            pltpu.SemaphoreType.DMA,
        ],
    )
    return f(head, relation, tail, ent_real, ent_imag, rel2)


# speedup vs baseline: 1.8964x; 1.0008x over previous
import jax
import jax.numpy as jnp
from jax import lax
from jax.experimental import pallas as pl
from jax.experimental.pallas import tpu as pltpu
from jax.experimental.pallas import tpu_sc as plsc

NUM_ENTITIES = 1000000
NUM_RELATIONS = 1000
EMBED_DIM = 64
BATCH = 16384

NC = 2
NS = 16
NW = NC * NS
B_PER_W = BATCH // NW
CHUNK = 128
NCHUNK = B_PER_W // CHUNK
L = 16
D2 = 2 * EMBED_DIM
RSLAB = 160


def _pair_rel_body(rrt, rit, rel2, br, bi, bc):
    wid = lax.axis_index("s") * NC + lax.axis_index("c")

    def combine(n):
        def row_body(r, carry):
            for j in range(EMBED_DIM // L):
                sl = pl.ds(j * L, L)
                bc[r, sl] = br[r, sl]
                bc[r, pl.ds(EMBED_DIM + j * L, L)] = bi[r, sl]
            return carry
        lax.fori_loop(0, n, row_body, 0)

    @pl.when(wid < 6)
    def _():
        rbase = wid * RSLAB
        pltpu.sync_copy(rrt.at[pl.ds(rbase, RSLAB)], br)
        pltpu.sync_copy(rit.at[pl.ds(rbase, RSLAB)], bi)
        combine(RSLAB)
        pltpu.sync_copy(bc, rel2.at[pl.ds(rbase, RSLAB)])

    @pl.when(wid == 6)
    def _():
        pltpu.sync_copy(rrt.at[pl.ds(960, 40)], br.at[pl.ds(0, 40)])
        pltpu.sync_copy(rit.at[pl.ds(960, 40)], bi.at[pl.ds(0, 40)])
        combine(40)
        pltpu.sync_copy(bc.at[pl.ds(0, 40)], rel2.at[pl.ds(960, 40)])


def _gather_body(head_r, rel_r, tail_r, er, ei, rel2, out_hbm,
                 idx_h, idx_r, idx_t, g_hr, g_hi, g_tr, g_ti, g_r,
                 out_v, sem, semr):
    wid = lax.axis_index("s") * NC + lax.axis_index("c")
    iota = lax.iota(jnp.int32, L)

    def chunk_body(ci, carry):
        base = wid * B_PER_W + ci * CHUNK
        pltpu.sync_copy(head_r.at[pl.ds(base, CHUNK)], idx_h)
        pltpu.sync_copy(rel_r.at[pl.ds(base, CHUNK)], idx_r)
        pltpu.sync_copy(tail_r.at[pl.ds(base, CHUNK)], idx_t)
        rel_cp = pltpu.async_copy(rel2.at[idx_r], g_r, semr)

        def fetch_body(g, carry2):
            vh = idx_h[pl.ds(g * L, L)]
            vt = idx_t[pl.ds(g * L, L)]
            cps = []
            for row in range(L):
                r = g * L + row
                ih = vh[row]
                it = vt[row]
                cps.append(pltpu.make_async_copy(er.at[ih], g_hr.at[r], sem))
                cps.append(pltpu.make_async_copy(ei.at[ih], g_hi.at[r], sem))
                cps.append(pltpu.make_async_copy(er.at[it], g_tr.at[r], sem))
                cps.append(pltpu.make_async_copy(ei.at[it], g_ti.at[r], sem))
            for cp in cps:
                cp.start()
            for _ in range(4):
                pltpu.make_async_copy(
                    er.at[pl.ds(0, L)], g_hr.at[pl.ds(0, L)], sem,
                ).wait()
            return carry2

        lax.fori_loop(0, CHUNK // L, fetch_body, 0)
        rel_cp.wait()

        def group_body(g, carry2):
            tot = jnp.zeros((L,), jnp.float32)
            for row in range(L):
                c = g * L + row
                acc = jnp.zeros((L,), jnp.float32)
                for j in range(EMBED_DIM // L):
                    sl = pl.ds(j * L, L)
                    hr = g_hr[c, sl]
                    hi = g_hi[c, sl]
                    tr = g_tr[c, sl]
                    ti = g_ti[c, sl]
                    rr = g_r[c, sl]
                    ri = g_r[c, pl.ds(EMBED_DIM + j * L, L)]
                    acc = acc + rr * (hr * tr + hi * ti) + ri * (hr * ti - hi * tr)
                s = lax.reduce_sum_p.bind(acc, axes=(0,))
                tot = jnp.where(iota == row, s, tot)
            out_v[pl.ds(g * L, L)] = tot
            return carry2

        lax.fori_loop(0, CHUNK // L, group_body, 0)
        pltpu.sync_copy(out_v, out_hbm.at[pl.ds(base, CHUNK)])
        return carry

    lax.fori_loop(0, NCHUNK, chunk_body, 0)


def kernel(head, relation, tail, ent_real, ent_imag, rel_real, rel_imag):
    mesh = plsc.VectorSubcoreMesh(core_axis_name="c", subcore_axis_name="s")
    cp_ = pltpu.CompilerParams(
        needs_layout_passes=False, use_tc_tiling_on_sc=True)
    pair = pl.kernel(
        _pair_rel_body,
        mesh=mesh,
        compiler_params=cp_,
        out_type=jax.ShapeDtypeStruct((NUM_RELATIONS, D2), jnp.float32),
        scratch_types=[
            pltpu.VMEM((RSLAB, EMBED_DIM), jnp.float32),
            pltpu.VMEM((RSLAB, EMBED_DIM), jnp.float32),
            pltpu.VMEM((RSLAB, D2), jnp.float32),
        ],
    )
    rel2 = pair(rel_real, rel_imag)
    f = pl.kernel(
        _gather_body,
        mesh=mesh,
        compiler_params=cp_,
        out_type=jax.ShapeDtypeStruct((BATCH,), jnp.float32),
        scratch_types=[
            pltpu.VMEM((CHUNK,), jnp.int32),
            pltpu.VMEM((CHUNK,), jnp.int32),
            pltpu.VMEM((CHUNK,), jnp.int32),
            pltpu.VMEM((CHUNK, EMBED_DIM), jnp.float32),
            pltpu.VMEM((CHUNK, EMBED_DIM), jnp.float32),
            pltpu.VMEM((CHUNK, EMBED_DIM), jnp.float32),
            pltpu.VMEM((CHUNK, EMBED_DIM), jnp.float32),
            pltpu.VMEM((CHUNK, D2), jnp.float32),
            pltpu.VMEM((CHUNK,), jnp.float32),
            pltpu.SemaphoreType.DMA,
            pltpu.SemaphoreType.DMA,
        ],
    )
    return f(head, relation, tail, ent_real, ent_imag, rel2)


# speedup vs baseline: 1.9168x; 1.0107x over previous
import jax
import jax.numpy as jnp
from jax import lax
from jax.experimental import pallas as pl
from jax.experimental.pallas import tpu as pltpu
from jax.experimental.pallas import tpu_sc as plsc

NUM_ENTITIES = 1000000
NUM_RELATIONS = 1000
EMBED_DIM = 64
BATCH = 16384

NC = 2
NS = 16
NW = NC * NS
B_PER_W = BATCH // NW
CHUNK = 128
NCHUNK = B_PER_W // CHUNK
L = 16
D2 = 2 * EMBED_DIM
RSLAB = 160


def _pair_rel_body(rrt, rit, rel2, br, bi, bc):
    wid = lax.axis_index("s") * NC + lax.axis_index("c")

    def combine(n):
        def row_body(r, carry):
            for j in range(EMBED_DIM // L):
                sl = pl.ds(j * L, L)
                bc[r, sl] = br[r, sl]
                bc[r, pl.ds(EMBED_DIM + j * L, L)] = bi[r, sl]
            return carry
        lax.fori_loop(0, n, row_body, 0)

    @pl.when(wid < 6)
    def _():
        rbase = wid * RSLAB
        pltpu.sync_copy(rrt.at[pl.ds(rbase, RSLAB)], br)
        pltpu.sync_copy(rit.at[pl.ds(rbase, RSLAB)], bi)
        combine(RSLAB)
        pltpu.sync_copy(bc, rel2.at[pl.ds(rbase, RSLAB)])

    @pl.when(wid == 6)
    def _():
        pltpu.sync_copy(rrt.at[pl.ds(960, 40)], br.at[pl.ds(0, 40)])
        pltpu.sync_copy(rit.at[pl.ds(960, 40)], bi.at[pl.ds(0, 40)])
        combine(40)
        pltpu.sync_copy(bc.at[pl.ds(0, 40)], rel2.at[pl.ds(960, 40)])


def _gather_body(head_r, rel_r, tail_r, er, ei, rel2, out_hbm,
                 idx_h, idx_r, idx_t, g_hr, g_hi, g_tr, g_ti, g_r,
                 out_v, sem, semr):
    wid = lax.axis_index("s") * NC + lax.axis_index("c")
    iota = lax.iota(jnp.int32, L)

    def chunk_body(ci, carry):
        base = wid * B_PER_W + ci * CHUNK
        pltpu.sync_copy(head_r.at[pl.ds(base, CHUNK)], idx_h)
        pltpu.sync_copy(rel_r.at[pl.ds(base, CHUNK)], idx_r)
        pltpu.sync_copy(tail_r.at[pl.ds(base, CHUNK)], idx_t)
        rel_cp = pltpu.async_copy(rel2.at[idx_r], g_r, semr)

        def fetch_body(g, carry2):
            cps = []
            for half in range(2):
                gg = 2 * g + half
                vh = idx_h[pl.ds(gg * L, L)]
                vt = idx_t[pl.ds(gg * L, L)]
                for row in range(L):
                    r = gg * L + row
                    ih = vh[row]
                    it = vt[row]
                    cps.append(pltpu.make_async_copy(er.at[ih], g_hr.at[r], sem))
                    cps.append(pltpu.make_async_copy(ei.at[ih], g_hi.at[r], sem))
                    cps.append(pltpu.make_async_copy(er.at[it], g_tr.at[r], sem))
                    cps.append(pltpu.make_async_copy(ei.at[it], g_ti.at[r], sem))
            for cp in cps:
                cp.start()
            for _ in range(4):
                pltpu.make_async_copy(
                    er.at[pl.ds(0, 2 * L)], g_hr.at[pl.ds(0, 2 * L)], sem,
                ).wait()
            return carry2

        lax.fori_loop(0, CHUNK // (2 * L), fetch_body, 0)
        rel_cp.wait()

        def group_body(g, carry2):
            tot = jnp.zeros((L,), jnp.float32)
            for row in range(L):
                c = g * L + row
                acc = jnp.zeros((L,), jnp.float32)
                for j in range(EMBED_DIM // L):
                    sl = pl.ds(j * L, L)
                    hr = g_hr[c, sl]
                    hi = g_hi[c, sl]
                    tr = g_tr[c, sl]
                    ti = g_ti[c, sl]
                    rr = g_r[c, sl]
                    ri = g_r[c, pl.ds(EMBED_DIM + j * L, L)]
                    acc = acc + rr * (hr * tr + hi * ti) + ri * (hr * ti - hi * tr)
                s = lax.reduce_sum_p.bind(acc, axes=(0,))
                tot = jnp.where(iota == row, s, tot)
            out_v[pl.ds(g * L, L)] = tot
            return carry2

        lax.fori_loop(0, CHUNK // L, group_body, 0)
        pltpu.sync_copy(out_v, out_hbm.at[pl.ds(base, CHUNK)])
        return carry

    lax.fori_loop(0, NCHUNK, chunk_body, 0)


def kernel(head, relation, tail, ent_real, ent_imag, rel_real, rel_imag):
    mesh = plsc.VectorSubcoreMesh(core_axis_name="c", subcore_axis_name="s")
    cp_ = pltpu.CompilerParams(
        needs_layout_passes=False, use_tc_tiling_on_sc=True)
    pair = pl.kernel(
        _pair_rel_body,
        mesh=mesh,
        compiler_params=cp_,
        out_type=jax.ShapeDtypeStruct((NUM_RELATIONS, D2), jnp.float32),
        scratch_types=[
            pltpu.VMEM((RSLAB, EMBED_DIM), jnp.float32),
            pltpu.VMEM((RSLAB, EMBED_DIM), jnp.float32),
            pltpu.VMEM((RSLAB, D2), jnp.float32),
        ],
    )
    rel2 = pair(rel_real, rel_imag)
    f = pl.kernel(
        _gather_body,
        mesh=mesh,
        compiler_params=cp_,
        out_type=jax.ShapeDtypeStruct((BATCH,), jnp.float32),
        scratch_types=[
            pltpu.VMEM((CHUNK,), jnp.int32),
            pltpu.VMEM((CHUNK,), jnp.int32),
            pltpu.VMEM((CHUNK,), jnp.int32),
            pltpu.VMEM((CHUNK, EMBED_DIM), jnp.float32),
            pltpu.VMEM((CHUNK, EMBED_DIM), jnp.float32),
            pltpu.VMEM((CHUNK, EMBED_DIM), jnp.float32),
            pltpu.VMEM((CHUNK, EMBED_DIM), jnp.float32),
            pltpu.VMEM((CHUNK, D2), jnp.float32),
            pltpu.VMEM((CHUNK,), jnp.float32),
            pltpu.SemaphoreType.DMA,
            pltpu.SemaphoreType.DMA,
        ],
    )
    return f(head, relation, tail, ent_real, ent_imag, rel2)
